# Initial kernel scaffold; baseline (speedup 1.0000x reference)
import jax
import jax.numpy as jnp
from jax.experimental import pallas as pl


def kernel(node_s, node_v, edge_s, edge_v, dist, vctr_norm, edge_index, params):
    raise NotImplementedError("write your pallas kernel here")


# speedup vs baseline: 9.3597x; 9.3597x over previous
import functools

import jax
import jax.numpy as jnp
from jax import lax
from jax.experimental import pallas as pl
from jax.experimental.pallas import tpu as pltpu
from jax.experimental.pallas import tpu_sc as plsc

N = 10000
E = 160000
F = 128
CUT = 5.0

NC = 2
NS = 16
NW = NC * NS

EW = E // NW
GC = 128
NFULL = EW // GC
TAIL0 = EW - GC


def _gather_body(ns_hbm, nv_hbm, src_hbm, dst_hbm, o_ns_s, o_ns_d, o_nv,
                 idx_s, idx_d, b_s, b_d, b_v, sem):
    wid = lax.axis_index("s") * NC + lax.axis_index("c")
    base = wid * EW
    pltpu.sync_copy(src_hbm.at[pl.ds(base, EW)], idx_s)
    pltpu.sync_copy(dst_hbm.at[pl.ds(base, EW)], idx_d)

    def chunk(s0):
        i1 = idx_s.at[pl.ds(s0, GC)]
        i2 = idx_d.at[pl.ds(s0, GC)]
        c1 = pltpu.async_copy(ns_hbm.at[i1], b_s, sem)
        c2 = pltpu.async_copy(ns_hbm.at[i2], b_d, sem)
        c3 = pltpu.async_copy(nv_hbm.at[i1], b_v, sem)
        c1.wait()
        c2.wait()
        c3.wait()
        pltpu.sync_copy(b_s, o_ns_s.at[pl.ds(base + s0, GC)])
        pltpu.sync_copy(b_d, o_ns_d.at[pl.ds(base + s0, GC)])
        pltpu.sync_copy(b_v, o_nv.at[pl.ds(base + s0, GC)])

    def loop_body(i, carry):
        chunk(i * GC)
        return carry

    lax.fori_loop(0, NFULL, loop_body, 0)
    chunk(TAIL0)


def _sc_gather(node_s, nv_flat, src, dst):
    mesh = plsc.VectorSubcoreMesh(core_axis_name="c", subcore_axis_name="s")
    return pl.kernel(
        _gather_body,
        out_type=(
            jax.ShapeDtypeStruct((E, F), jnp.float32),
            jax.ShapeDtypeStruct((E, F), jnp.float32),
            jax.ShapeDtypeStruct((E, 3 * F), jnp.float32),
        ),
        mesh=mesh,
        scratch_types=[
            pltpu.VMEM((EW,), jnp.int32),
            pltpu.VMEM((EW,), jnp.int32),
            pltpu.VMEM((GC, F), jnp.float32),
            pltpu.VMEM((GC, F), jnp.float32),
            pltpu.VMEM((GC, 3 * F), jnp.float32),
            pltpu.SemaphoreType.DMA,
        ],
    )(node_s, nv_flat, src, dst)


ET = E // NS
SUB = 80
NSUB = ET // SUB
NPAD = 10240
RT = NPAD // NS
ZR = 64


def _scatter_body(esu, evu0, evu1, evu2, dst3d, zeros_h, ones_h,
                  out_s, out_v0, out_v1, out_v2, out_c0, out_c1,
                  zbuf, obuf, idxb, rowb, acc, sem):
    del sem
    cid = lax.axis_index("c")
    sid = lax.axis_index("s")
    r0 = sid * RT
    pltpu.sync_copy(zeros_h, zbuf)
    pltpu.sync_copy(ones_h, obuf)
    pltpu.sync_copy(dst3d.at[sid], idxb)

    def zero_stripe():
        for j in range(RT // ZR):
            pltpu.sync_copy(zbuf, acc.at[pl.ds(r0 + j * ZR, ZR)])

    for t in range(2):
        zero_stripe()
        plsc.subcore_barrier()

        src_c0 = (esu, evu0)[t]
        src_c1 = (evu1, evu2)[t]

        def blk_body(i, carry):
            e0 = sid * ET + i * SUB

            @pl.when(cid == 0)
            def _():
                pltpu.sync_copy(src_c0.at[pl.ds(e0, SUB)], rowb)

            @pl.when(cid == 1)
            def _():
                pltpu.sync_copy(src_c1.at[pl.ds(e0, SUB)], rowb)

            pltpu.sync_copy(rowb, acc.at[idxb.at[i]], add=True)
            return carry

        lax.fori_loop(0, NSUB, blk_body, 0)
        plsc.subcore_barrier()

        dst_c0 = (out_s, out_v0)[t]
        dst_c1 = (out_v1, out_v2)[t]

        @pl.when(cid == 0)
        def _():
            pltpu.sync_copy(acc.at[pl.ds(r0, RT)], dst_c0.at[pl.ds(r0, RT)])

        @pl.when(cid == 1)
        def _():
            pltpu.sync_copy(acc.at[pl.ds(r0, RT)], dst_c1.at[pl.ds(r0, RT)])

    zero_stripe()
    plsc.subcore_barrier()

    @pl.when(cid == 0)
    def _():
        def cnt_body(i, carry):
            pltpu.sync_copy(obuf, acc.at[idxb.at[2 * i]], add=True)
            return carry
        lax.fori_loop(0, (NSUB + 1) // 2, cnt_body, 0)

    @pl.when(cid == 1)
    def _():
        def cnt_body(i, carry):
            pltpu.sync_copy(obuf, acc.at[idxb.at[2 * i + 1]], add=True)
            return carry
        lax.fori_loop(0, NSUB // 2, cnt_body, 0)

    plsc.subcore_barrier()

    @pl.when(cid == 0)
    def _():
        pltpu.sync_copy(acc.at[pl.ds(r0, RT)], out_c0.at[pl.ds(r0, RT)])

    @pl.when(cid == 1)
    def _():
        pltpu.sync_copy(acc.at[pl.ds(r0, RT)], out_c1.at[pl.ds(r0, RT)])


def _sc_scatter(esu, evu0, evu1, evu2, dst3d):
    mesh = plsc.VectorSubcoreMesh(core_axis_name="c", subcore_axis_name="s")
    zeros_h = jnp.zeros((ZR, F), jnp.float32)
    ones_h = jnp.ones((SUB, F), jnp.float32)
    nf = jax.ShapeDtypeStruct((NPAD, F), jnp.float32)
    return pl.kernel(
        _scatter_body,
        out_type=(nf, nf, nf, nf, nf, nf),
        mesh=mesh,
        scratch_types=[
            pltpu.VMEM((ZR, F), jnp.float32),
            pltpu.VMEM((SUB, F), jnp.float32),
            pltpu.VMEM((NSUB, SUB), jnp.int32),
            pltpu.VMEM((SUB, F), jnp.float32),
            pltpu.VMEM_SHARED((NPAD, F), jnp.float32),
            pltpu.SemaphoreType.DMA,
        ],
    )(esu, evu0, evu1, evu2, dst3d, zeros_h, ones_h)


EB = 640


def _edge_body(ns_s, ns_d, es, nvs, ev, d, v0, v1, v2,
               wnna, wnnb, bnn, wep, bep, wg1, bg1, wgg, bgg, wg2, bg2,
               wev, bev,
               esu_o, ev0_o, ev1_o, ev2_o, eso_o, evo_o):
    dot = lambda a, b: lax.dot_general(a, b, (((1,), (0,)), ((), ())),
                                       preferred_element_type=jnp.float32)
    nn = dot(ns_s[...], wnna[...]) + dot(ns_d[...], wnnb[...]) + bnn[...]
    em = nn * (dot(es[...], wep[...]) + bep[...])
    h = jax.nn.silu(dot(em, wg1[...]) + bg1[...])
    gate = jax.nn.sigmoid(dot(em, wgg[...]) + bgg[...])
    mess = (dot(h, wg2[...]) + bg2[...]) * gate
    dd = d[...]
    cut = 0.5 * (jnp.cos(jnp.pi / CUT * dd) + 1.0)
    cut = cut * (dd < CUT).astype(jnp.float32)
    esu = mess * cut
    vc = dot(esu, wev[...]) + bev[...]
    nc, ec, rc = vc[:, :F], vc[:, F:2 * F], vc[:, 2 * F:]
    evb = ev[...]
    nvb = nvs[...]
    vs = (v0[...], v1[...], v2[...])
    evo = []
    for k, outk in enumerate((ev0_o, ev1_o, ev2_o)):
        evk = (nvb[:, k * F:(k + 1) * F] * nc
               + evb[:, k * F:(k + 1) * F] * ec
               + vs[k] * rc) * cut
        outk[...] = evk
        evo.append(evk + evb[:, k * F:(k + 1) * F])
    esu_o[...] = esu
    eso_o[...] = esu + es[...]
    evo_o[...] = jnp.concatenate(evo, axis=1)


def _tc_edge(ns_src, ns_dst, edge_s, nv_src, ev_flat, dist2, v0, v1, v2, p):
    eb = lambda w: pl.BlockSpec((EB, w), lambda i: (i, 0))
    wb = lambda a: pl.BlockSpec(a.shape, lambda i: (0,) * a.ndim)
    wnna = p['W_nn'][:F]
    wnnb = p['W_nn'][F:]
    weights = (wnna, wnnb, p['b_nn'].reshape(1, F),
               p['W_ep'], p['b_ep'].reshape(1, F),
               p['W_g1'], p['b_g1'].reshape(1, F),
               p['W_gg'], p['b_gg'].reshape(1, F),
               p['W_g2'], p['b_g2'].reshape(1, F),
               p['W_ev'], p['b_ev'].reshape(1, 3 * F))
    ef = jax.ShapeDtypeStruct((E, F), jnp.float32)
    return pl.pallas_call(
        _edge_body,
        grid=(E // EB,),
        in_specs=[eb(F), eb(F), eb(F), eb(3 * F), eb(3 * F),
                  eb(1), eb(1), eb(1), eb(1)] + [wb(w) for w in weights],
        out_specs=[eb(F), eb(F), eb(F), eb(F), eb(F), eb(3 * F)],
        out_shape=[ef, ef, ef, ef, ef,
                   jax.ShapeDtypeStruct((E, 3 * F), jnp.float32)],
    )(ns_src, ns_dst, edge_s, nv_src, ev_flat, dist2, v0, v1, v2, *weights)


NB = 1000


def _node_body(sa, va0, va1, va2, cnt0, cnt1, ns, nv,
               wnvout, wnvca, wnvcb, bnvc, wnvp, wnsp, bnsp,
               lng, lnb, scl,
               so_o, vo_o):
    dot = lambda a, b: lax.dot_general(a, b, (((1,), (0,)), ((), ())),
                                       preferred_element_type=jnp.float32)
    c = jnp.maximum(cnt0[...][:, 0:1] + cnt1[...][:, 0:1], 1.0)
    inv = 1.0 / c
    nes = sa[...] * inv
    nevs = (va0[...] * inv, va1[...] * inv, va2[...] * inv)
    nvw = [dot(nevs[k], wnvout[...]) for k in range(3)]
    o1 = [w[:, :F] for w in nvw]
    o2 = [w[:, F:2 * F] for w in nvw]
    o3 = [w[:, 2 * F:] for w in nvw]
    norm3 = jnp.sqrt(o3[0] * o3[0] + o3[1] * o3[1] + o3[2] * o3[2])
    vch = dot(nes, wnvca[...]) + dot(norm3, wnvcb[...]) + bnvc[...]
    nvu = [o1[k] * vch + o2[k] for k in range(3)]
    pv = [dot(nvu[k], wnvp[...]) for k in range(3)]
    nvdot = sum(p[:, :F] * p[:, F:] for p in pv)
    sp = jax.nn.silu(dot(nes, wnsp[...]) + bnsp[...])
    nsu = nvdot * sp[:, :F] + sp[:, F:]
    so = nsu + ns[...]
    mu = jnp.mean(so, axis=-1, keepdims=True)
    var = jnp.mean((so - mu) ** 2, axis=-1, keepdims=True)
    so_o[...] = (so - mu) * lax.rsqrt(var + 1e-5) * lng[...] + lnb[...]
    nvb = nv[...]
    nvo = [nvu[k] + nvb[:, k * F:(k + 1) * F] for k in range(3)]
    vn = jnp.sqrt(nvo[0] * nvo[0] + nvo[1] * nvo[1] + nvo[2] * nvo[2])
    r = scl[...] / (vn + 1e-8)
    vo_o[...] = jnp.concatenate([nvo[k] * r for k in range(3)], axis=1)


def _tc_node(s_acc, v0, v1, v2, cnt0, cnt1, node_s, nv_flat, p):
    nb = lambda w: pl.BlockSpec((NB, w), lambda i: (i, 0))
    wb = lambda a: pl.BlockSpec(a.shape, lambda i: (0,) * a.ndim)
    weights = (p['W_nvout'], p['W_nvc'][:F], p['W_nvc'][F:],
               p['b_nvc'].reshape(1, F), p['W_nvp'], p['W_nsp'],
               p['b_nsp'].reshape(1, 2 * F), p['ln_g'].reshape(1, F),
               p['ln_b'].reshape(1, F), p['scale_nv'].reshape(1, F))
    return pl.pallas_call(
        _node_body,
        grid=(N // NB,),
        in_specs=[nb(F), nb(F), nb(F), nb(F), nb(F), nb(F), nb(F), nb(3 * F)]
                 + [wb(w) for w in weights],
        out_specs=[nb(F), nb(3 * F)],
        out_shape=[jax.ShapeDtypeStruct((N, F), jnp.float32),
                   jax.ShapeDtypeStruct((N, 3 * F), jnp.float32)],
    )(s_acc, v0, v1, v2, cnt0, cnt1, node_s, nv_flat, *weights)


def kernel(node_s, node_v, edge_s, edge_v, dist, vctr_norm, edge_index, params):
    src = edge_index[0]
    dst = edge_index[1]
    nv_flat = node_v.reshape(N, 3 * F)
    ev_flat = edge_v.reshape(E, 3 * F)
    dist2 = dist.reshape(E, 1)
    v0 = vctr_norm[:, 0:1]
    v1 = vctr_norm[:, 1:2]
    v2 = vctr_norm[:, 2:3]

    ns_src, ns_dst, nv_src = _sc_gather(node_s, nv_flat, src, dst)
    esu, ev0, ev1, ev2, eso, evo = _tc_edge(
        ns_src, ns_dst, edge_s, nv_src, ev_flat, dist2, v0, v1, v2, params)
    s_acc, a0, a1, a2, c0, c1 = _sc_scatter(esu, ev0, ev1, ev2,
                                            dst.reshape(NS, NSUB, SUB))
    so, vo = _tc_node(s_acc, a0, a1, a2, c0, c1, node_s, nv_flat, params)
    return so, vo.reshape(N, 3, F), eso, evo.reshape(E, 3, F)
